# baseline (speedup 1.0000x reference)
import functools

import jax
import jax.numpy as jnp
from jax import lax
from jax.experimental import pallas as pl
from jax.experimental.pallas import tpu as pltpu
from jax.experimental.pallas import tpu_sc as plsc

N = 10000
E = 320000
D = 128
HD = D // 2
G = 64

NC = 2
NS = 16
NW = NC * NS
EPW = E // NW
CHUNK = 80
NCHUNK = 125
EPWP = NCHUNK * CHUNK
NP = 10240
TRASH = NP - 1
RPT = NP // NS
ZR = 128
NZ = RPT // ZR
NRING = 10
NDSEM = 8

_mesh = plsc.VectorSubcoreMesh(core_axis_name="c", subcore_axis_name="s")


@functools.partial(
    pl.kernel,
    mesh=_mesh,
    out_type=jax.ShapeDtypeStruct((NC, NP, 16), jnp.float32),
    scratch_types=(
        [
            pltpu.VMEM((NCHUNK, CHUNK), jnp.int32),
            pltpu.VMEM((CHUNK, 16), jnp.float32),
            pltpu.VMEM((RPT, 16), jnp.float32),
            pltpu.VMEM_SHARED((NP, 16), jnp.float32),
        ]
        + [pltpu.SemaphoreType.DMA] * NDSEM
    ),
    compiler_params=pltpu.CompilerParams(use_tc_tiling_on_sc=False),
)
def _sc_degree(dst_hbm, ones_hbm, zeros_hbm, degp_hbm, idx_v, ones_v, stage_v, acc,
               *sems):
    cid = lax.axis_index("c")
    sid = lax.axis_index("s")
    wid = cid * NS + sid
    pltpu.sync_copy(dst_hbm.at[wid], idx_v)
    pltpu.sync_copy(ones_hbm, ones_v)
    pltpu.sync_copy(zeros_hbm, stage_v)
    pltpu.sync_copy(stage_v, acc.at[pl.ds(sid * RPT, RPT)])
    plsc.subcore_barrier()

    def s_issue(j, r):
        pltpu.async_copy(ones_v, acc.at[idx_v.at[j]], sems[r], add=True)

    def s_wait(j, r):
        pltpu.make_async_copy(ones_v, acc.at[idx_v.at[j]], sems[r]).wait()

    for j in range(NDSEM):
        s_issue(j, j)

    def outer(o, carry):
        for r in range(NDSEM):
            j = NDSEM * o + r
            s_wait(j - NDSEM, r)
            s_issue(j, r)
        return carry

    lax.fori_loop(1, NCHUNK // NDSEM, outer, 0)
    for j in range(NCHUNK - NCHUNK % NDSEM, NCHUNK):
        s_wait(j - NDSEM, j % NDSEM)
        s_issue(j, j % NDSEM)
    for r in range(NDSEM):
        s_wait(0, r)

    plsc.subcore_barrier()
    pltpu.sync_copy(acc.at[pl.ds(sid * RPT, RPT)], stage_v)
    pltpu.sync_copy(stage_v, degp_hbm.at[cid, pl.ds(sid * RPT, RPT)])


@functools.partial(
    pl.kernel,
    mesh=_mesh,
    out_type=[
        jax.ShapeDtypeStruct((NC, NP, HD), jnp.float32),
        jax.ShapeDtypeStruct((NC, NP, HD), jnp.float32),
    ],
    scratch_types=(
        [
            pltpu.VMEM((NCHUNK, CHUNK), jnp.int32),
            pltpu.VMEM((NCHUNK, CHUNK), jnp.int32),
            pltpu.VMEM((ZR, HD), jnp.float32),
            pltpu.VMEM((ZR, HD), jnp.float32),
            pltpu.VMEM_SHARED((NP, HD), jnp.float32),
        ]
        + [pltpu.VMEM((CHUNK, HD), jnp.float32)] * NRING
        + [pltpu.SemaphoreType.DMA] * (2 * NRING + 3)
    ),
    compiler_params=pltpu.CompilerParams(use_tc_tiling_on_sc=False),
)
def _sc_scatter(za_hbm, zb_hbm, src_hbm, dst_hbm, zeros_hbm, spa_hbm, spb_hbm,
                src_v, dst_v, stage_v, stage2_v, acc, *ring):
    bufs = ring[:NRING]
    gsem = ring[NRING:2 * NRING]
    ssem = ring[2 * NRING:3 * NRING]
    zsem = ring[3 * NRING]
    wsem = ring[3 * NRING + 1:]
    stg = (stage_v, stage2_v)
    cid = lax.axis_index("c")
    sid = lax.axis_index("s")
    wid = cid * NS + sid
    pltpu.sync_copy(src_hbm.at[wid], src_v)
    pltpu.sync_copy(dst_hbm.at[wid], dst_v)
    LA = NRING // 2

    for z_hbm, sp_hbm in ((za_hbm, spa_hbm), (zb_hbm, spb_hbm)):
        pltpu.sync_copy(zeros_hbm, stage_v)
        for k in range(NZ):
            pltpu.async_copy(stage_v, acc.at[pl.ds(sid * RPT + k * ZR, ZR)], zsem)
        for k in range(NZ):
            pltpu.make_async_copy(
                stage_v, acc.at[pl.ds(sid * RPT + k * ZR, ZR)], zsem).wait()
        plsc.subcore_barrier()

        def g_issue(j, b):
            pltpu.async_copy(z_hbm.at[src_v.at[j]], bufs[b], gsem[b])

        def g_wait(j, b):
            pltpu.make_async_copy(z_hbm.at[src_v.at[j]], bufs[b], gsem[b]).wait()

        def s_issue(j, b):
            pltpu.async_copy(bufs[b], acc.at[dst_v.at[j]], ssem[b], add=True)

        def s_wait(j, b):
            pltpu.make_async_copy(bufs[b], acc.at[dst_v.at[j]], ssem[b]).wait()

        for j in range(LA):
            g_issue(j, j)
        for j in range(NRING):
            b, b2 = j, (j + LA) % NRING
            if j >= NRING - LA:
                s_wait(j + LA - NRING, b2)
            g_issue(j + LA, b2)
            g_wait(j, b)
            s_issue(j, b)

        def outer(o, carry):
            for r in range(NRING):
                j = NRING * o + r
                b2 = (r + LA) % NRING
                s_wait(j + LA - NRING, b2)
                g_issue(j + LA, b2)
                g_wait(j, r)
                s_issue(j, r)
            return carry

        lax.fori_loop(1, NCHUNK // NRING, outer, 0)

        for j in range(NCHUNK - LA, NCHUNK):
            b = j % NRING
            s_wait(j + LA - NRING, (j + LA) % NRING)
            g_wait(j, b)
            s_issue(j, b)
        for k in range(NCHUNK + LA - NRING, NCHUNK):
            s_wait(k, k % NRING)

        plsc.subcore_barrier()
        for k in range(NZ):
            b = k % 2
            if k >= 2:
                pltpu.make_async_copy(
                    stg[b],
                    sp_hbm.at[cid, pl.ds(sid * RPT + (k - 2) * ZR, ZR)],
                    wsem[b]).wait()
            pltpu.sync_copy(acc.at[pl.ds(sid * RPT + k * ZR, ZR)], stg[b])
            pltpu.async_copy(
                stg[b], sp_hbm.at[cid, pl.ds(sid * RPT + k * ZR, ZR)], wsem[b])
        for k in (NZ - 2, NZ - 1):
            pltpu.make_async_copy(
                stg[k % 2], sp_hbm.at[cid, pl.ds(sid * RPT + k * ZR, ZR)],
                wsem[k % 2]).wait()
        plsc.subcore_barrier()


_R = 1000
_NB = N // _R


def _tc_mm_body(x_ref, w_ref, xw_ref):
    xw_ref[...] = jnp.dot(x_ref[...], w_ref[...],
                          preferred_element_type=jnp.float32)


def _tc_prescale_body(degp_ref, xw_ref, za_ref, zb_ref, dinv_ref):
    deg = degp_ref[0, :, 0:1] + degp_ref[1, :, 0:1] + 1.0
    dinv = lax.rsqrt(deg)
    z = xw_ref[...] * dinv
    za_ref[...] = z[:, :HD]
    zb_ref[...] = z[:, HD:]
    dinv_ref[...] = jnp.broadcast_to(dinv, (_R, 8))


def _combine(spa_ref, spb_ref, za_ref, zb_ref, dinv, b_ref):
    ha = dinv * (spa_ref[0] + spa_ref[1] + za_ref[...]) + b_ref[:, :HD]
    hb = dinv * (spb_ref[0] + spb_ref[1] + zb_ref[...]) + b_ref[:, HD:]
    return jnp.concatenate([ha, hb], axis=1)


def _tc_layer_body(spa_ref, spb_ref, za_ref, zb_ref, dinv_ref, b_ref, w_ref,
                   z2a_ref, z2b_ref):
    dinv = dinv_ref[:, 0:1]
    h = jnp.maximum(_combine(spa_ref, spb_ref, za_ref, zb_ref, dinv, b_ref), 0.0)
    z2 = dinv * jnp.dot(h, w_ref[...], preferred_element_type=jnp.float32)
    z2a_ref[...] = z2[:, :HD]
    z2b_ref[...] = z2[:, HD:]


def _tc_head_body(spa_ref, spb_ref, za_ref, zb_ref, dinv_ref, b_ref, batch_ref,
                  fc1w_ref, fc1b_ref, fc2w_ref, fc2b_ref,
                  out_ref, pool_acc, cnt_acc):
    i = pl.program_id(0)

    @pl.when(i == 0)
    def _():
        pool_acc[...] = jnp.zeros((G, D), jnp.float32)
        cnt_acc[...] = jnp.zeros((G, 8), jnp.float32)

    dinv = dinv_ref[:, 0:1]
    h2 = _combine(spa_ref, spb_ref, za_ref, zb_ref, dinv, b_ref)
    b = batch_ref[0, 0, :]
    oh = (lax.broadcasted_iota(jnp.int32, (G, _R), 0) == b[None, :]
          ).astype(jnp.float32)
    pool_acc[...] += jnp.dot(oh, h2, preferred_element_type=jnp.float32)
    cnt_acc[...] += jnp.broadcast_to(jnp.sum(oh, axis=1, keepdims=True), (G, 8))

    @pl.when(i == _NB - 1)
    def _():
        p = pool_acc[...] / jnp.maximum(cnt_acc[:, 0:1], 1.0)
        p = jnp.dot(p, fc1w_ref[...], preferred_element_type=jnp.float32)
        p = jnp.maximum(p + fc1b_ref[...], 0.0)
        p = jnp.dot(p, fc2w_ref[...], preferred_element_type=jnp.float32)
        out_ref[...] = p + fc2b_ref[...]


def _row_spec(shape):
    return pl.BlockSpec(shape, lambda i: (i, 0))


def _sp_spec():
    return pl.BlockSpec((NC, _R, HD), lambda i: (0, i, 0))


def _full_spec(shape):
    return pl.BlockSpec(shape, lambda i: tuple(0 for _ in shape))


def kernel(x, edge_index, batch, W1, b1, W2, b2, fc1_W, fc1_b, fc2_W, fc2_b):
    src = edge_index[0].reshape(NW, NCHUNK, CHUNK)
    dst = edge_index[1].reshape(NW, NCHUNK, CHUNK)
    ones8 = jnp.ones((CHUNK, 16), jnp.float32)
    zeros8 = jnp.zeros((RPT, 16), jnp.float32)
    zerosh = jnp.zeros((ZR, HD), jnp.float32)

    degp = _sc_degree(dst, ones8, zeros8)

    xw1 = pl.pallas_call(
        _tc_mm_body,
        grid=(_NB,),
        in_specs=[_row_spec((_R, D)), _full_spec((D, D))],
        out_specs=_row_spec((_R, D)),
        out_shape=jax.ShapeDtypeStruct((N, D), jnp.float32),
    )(x, W1)

    z1a, z1b, dinv8 = pl.pallas_call(
        _tc_prescale_body,
        grid=(_NB,),
        in_specs=[
            pl.BlockSpec((NC, _R, 16), lambda i: (0, i, 0)),
            _row_spec((_R, D)),
        ],
        out_specs=[_row_spec((_R, HD)), _row_spec((_R, HD)), _row_spec((_R, 8))],
        out_shape=[
            jax.ShapeDtypeStruct((N, HD), jnp.float32),
            jax.ShapeDtypeStruct((N, HD), jnp.float32),
            jax.ShapeDtypeStruct((N, 8), jnp.float32),
        ],
    )(degp, xw1)

    sp1a, sp1b = _sc_scatter(z1a, z1b, src, dst, zerosh)

    z2a, z2b = pl.pallas_call(
        _tc_layer_body,
        grid=(_NB,),
        in_specs=[
            _sp_spec(),
            _sp_spec(),
            _row_spec((_R, HD)),
            _row_spec((_R, HD)),
            _row_spec((_R, 8)),
            _full_spec((1, D)),
            _full_spec((D, D)),
        ],
        out_specs=[_row_spec((_R, HD)), _row_spec((_R, HD))],
        out_shape=[
            jax.ShapeDtypeStruct((N, HD), jnp.float32),
            jax.ShapeDtypeStruct((N, HD), jnp.float32),
        ],
    )(sp1a, sp1b, z1a, z1b, dinv8, b1.reshape(1, D), W2)

    sp2a, sp2b = _sc_scatter(z2a, z2b, src, dst, zerosh)

    out = pl.pallas_call(
        _tc_head_body,
        grid=(_NB,),
        in_specs=[
            _sp_spec(),
            _sp_spec(),
            _row_spec((_R, HD)),
            _row_spec((_R, HD)),
            _row_spec((_R, 8)),
            _full_spec((1, D)),
            pl.BlockSpec((1, 1, _R), lambda i: (i, 0, 0)),
            _full_spec((D, D)),
            _full_spec((1, D)),
            _full_spec((D, D)),
            _full_spec((1, D)),
        ],
        out_specs=_full_spec((G, D)),
        out_shape=jax.ShapeDtypeStruct((G, D), jnp.float32),
        scratch_shapes=[
            pltpu.VMEM((G, D), jnp.float32),
            pltpu.VMEM((G, 8), jnp.float32),
        ],
    )(sp2a, sp2b, z2a, z2b, dinv8, b2.reshape(1, D), batch.reshape(_NB, 1, _R),
      fc1_W, fc1_b.reshape(1, D), fc2_W, fc2_b.reshape(1, D))

    return out


# speedup vs baseline: 1.0033x; 1.0033x over previous
import functools

import jax
import jax.numpy as jnp
from jax import lax
from jax.experimental import pallas as pl
from jax.experimental.pallas import tpu as pltpu
from jax.experimental.pallas import tpu_sc as plsc

N = 10000
E = 320000
D = 128
HD = D // 2
G = 64

NC = 2
NS = 16
NW = NC * NS
EPW = E // NW
CHUNK = 80
NCHUNK = 125
EPWP = NCHUNK * CHUNK
NP = 10240
TRASH = NP - 1
RPT = NP // NS
ZR = 128
NZ = RPT // ZR
NRING = 10
NDSEM = 8

_mesh = plsc.VectorSubcoreMesh(core_axis_name="c", subcore_axis_name="s")


@functools.partial(
    pl.kernel,
    mesh=_mesh,
    out_type=jax.ShapeDtypeStruct((NC, NP, 16), jnp.float32),
    scratch_types=(
        [
            pltpu.VMEM((NCHUNK, CHUNK), jnp.int32),
            pltpu.VMEM((CHUNK, 16), jnp.float32),
            pltpu.VMEM((RPT, 16), jnp.float32),
            pltpu.VMEM_SHARED((NP, 16), jnp.float32),
        ]
        + [pltpu.SemaphoreType.DMA] * NDSEM
    ),
    compiler_params=pltpu.CompilerParams(use_tc_tiling_on_sc=False),
)
def _sc_degree(dst_hbm, ones_hbm, zeros_hbm, degp_hbm, idx_v, ones_v, stage_v, acc,
               *sems):
    cid = lax.axis_index("c")
    sid = lax.axis_index("s")
    wid = cid * NS + sid
    pltpu.sync_copy(dst_hbm.at[wid], idx_v)
    pltpu.sync_copy(ones_hbm, ones_v)
    pltpu.sync_copy(zeros_hbm, stage_v)
    pltpu.sync_copy(stage_v, acc.at[pl.ds(sid * RPT, RPT)])
    plsc.subcore_barrier()

    def s_issue(j, r):
        pltpu.async_copy(ones_v, acc.at[idx_v.at[j]], sems[r], add=True)

    def s_wait(j, r):
        pltpu.make_async_copy(ones_v, acc.at[idx_v.at[j]], sems[r]).wait()

    for j in range(NDSEM):
        s_issue(j, j)

    def outer(o, carry):
        for r in range(NDSEM):
            j = NDSEM * o + r
            s_wait(j - NDSEM, r)
            s_issue(j, r)
        return carry

    lax.fori_loop(1, NCHUNK // NDSEM, outer, 0)
    for j in range(NCHUNK - NCHUNK % NDSEM, NCHUNK):
        s_wait(j - NDSEM, j % NDSEM)
        s_issue(j, j % NDSEM)
    for r in range(NDSEM):
        s_wait(0, r)

    plsc.subcore_barrier()
    pltpu.sync_copy(acc.at[pl.ds(sid * RPT, RPT)], stage_v)
    pltpu.sync_copy(stage_v, degp_hbm.at[cid, pl.ds(sid * RPT, RPT)])


@functools.partial(
    pl.kernel,
    mesh=_mesh,
    out_type=[
        jax.ShapeDtypeStruct((NC, NP, HD), jnp.float32),
        jax.ShapeDtypeStruct((NC, NP, HD), jnp.float32),
    ],
    scratch_types=(
        [
            pltpu.VMEM((NCHUNK, CHUNK), jnp.int32),
            pltpu.VMEM((NCHUNK, CHUNK), jnp.int32),
            pltpu.VMEM((ZR, HD), jnp.float32),
            pltpu.VMEM((ZR, HD), jnp.float32),
            pltpu.VMEM_SHARED((NP, HD), jnp.float32),
        ]
        + [pltpu.VMEM((CHUNK, HD), jnp.float32)] * NRING
        + [pltpu.SemaphoreType.DMA] * (2 * NRING + 3)
    ),
    compiler_params=pltpu.CompilerParams(use_tc_tiling_on_sc=False),
)
def _sc_scatter(za_hbm, zb_hbm, src_hbm, dst_hbm, zeros_hbm, spa_hbm, spb_hbm,
                src_v, dst_v, stage_v, stage2_v, acc, *ring):
    bufs = ring[:NRING]
    gsem = ring[NRING:2 * NRING]
    ssem = ring[2 * NRING:3 * NRING]
    zsem = ring[3 * NRING]
    wsem = ring[3 * NRING + 1:]
    stg = (stage_v, stage2_v)
    cid = lax.axis_index("c")
    sid = lax.axis_index("s")
    wid = cid * NS + sid
    pltpu.sync_copy(src_hbm.at[wid], src_v)
    pltpu.sync_copy(dst_hbm.at[wid], dst_v)
    LA = NRING // 2

    for z_hbm, sp_hbm in ((za_hbm, spa_hbm), (zb_hbm, spb_hbm)):
        pltpu.sync_copy(zeros_hbm, stage_v)
        for k in range(NZ):
            pltpu.async_copy(stage_v, acc.at[pl.ds(sid * RPT + k * ZR, ZR)], zsem)
        for k in range(NZ):
            pltpu.make_async_copy(
                stage_v, acc.at[pl.ds(sid * RPT + k * ZR, ZR)], zsem).wait()
        plsc.subcore_barrier()

        def g_issue(j, b):
            pltpu.async_copy(z_hbm.at[src_v.at[j]], bufs[b], gsem[b])

        def g_wait(j, b):
            pltpu.make_async_copy(z_hbm.at[src_v.at[j]], bufs[b], gsem[b]).wait()

        def s_issue(j, b):
            pltpu.async_copy(bufs[b], acc.at[dst_v.at[j]], ssem[b], add=True)

        def s_wait(j, b):
            pltpu.make_async_copy(bufs[b], acc.at[dst_v.at[j]], ssem[b]).wait()

        for j in range(LA):
            g_issue(j, j)
        for j in range(NRING):
            b, b2 = j, (j + LA) % NRING
            if j >= NRING - LA:
                s_wait(j + LA - NRING, b2)
            g_issue(j + LA, b2)
            g_wait(j, b)
            s_issue(j, b)

        def outer(o, carry):
            for r in range(NRING):
                j = NRING * o + r
                b2 = (r + LA) % NRING
                s_wait(j + LA - NRING, b2)
                g_issue(j + LA, b2)
                g_wait(j, r)
                s_issue(j, r)
            return carry

        lax.fori_loop(1, NCHUNK // NRING, outer, 0)

        for j in range(NCHUNK - LA, NCHUNK):
            b = j % NRING
            s_wait(j + LA - NRING, (j + LA) % NRING)
            g_wait(j, b)
            s_issue(j, b)
        for k in range(NCHUNK + LA - NRING, NCHUNK):
            s_wait(k, k % NRING)

        plsc.subcore_barrier()
        for k in range(NZ):
            b = k % 2
            if k >= 2:
                pltpu.make_async_copy(
                    stg[b],
                    sp_hbm.at[cid, pl.ds(sid * RPT + (k - 2) * ZR, ZR)],
                    wsem[b]).wait()
            pltpu.sync_copy(acc.at[pl.ds(sid * RPT + k * ZR, ZR)], stg[b])
            pltpu.async_copy(
                stg[b], sp_hbm.at[cid, pl.ds(sid * RPT + k * ZR, ZR)], wsem[b])
        for k in (NZ - 2, NZ - 1):
            pltpu.make_async_copy(
                stg[k % 2], sp_hbm.at[cid, pl.ds(sid * RPT + k * ZR, ZR)],
                wsem[k % 2]).wait()
        plsc.subcore_barrier()


_R = 1000
_NB = N // _R


def _tc_prescale_body(degp_ref, x_ref, w_ref, za_ref, zb_ref, dinv_ref):
    deg = degp_ref[0, :, 0:1] + degp_ref[1, :, 0:1] + 1.0
    dinv = lax.rsqrt(deg)
    xw = jnp.dot(x_ref[...], w_ref[...], preferred_element_type=jnp.float32)
    z = xw * dinv
    za_ref[...] = z[:, :HD]
    zb_ref[...] = z[:, HD:]
    dinv_ref[...] = jnp.broadcast_to(dinv, (_R, 8))


def _combine(spa_ref, spb_ref, za_ref, zb_ref, dinv, b_ref):
    ha = dinv * (spa_ref[0] + spa_ref[1] + za_ref[...]) + b_ref[:, :HD]
    hb = dinv * (spb_ref[0] + spb_ref[1] + zb_ref[...]) + b_ref[:, HD:]
    return jnp.concatenate([ha, hb], axis=1)


def _tc_layer_body(spa_ref, spb_ref, za_ref, zb_ref, dinv_ref, b_ref, w_ref,
                   z2a_ref, z2b_ref):
    dinv = dinv_ref[:, 0:1]
    h = jnp.maximum(_combine(spa_ref, spb_ref, za_ref, zb_ref, dinv, b_ref), 0.0)
    z2 = dinv * jnp.dot(h, w_ref[...], preferred_element_type=jnp.float32)
    z2a_ref[...] = z2[:, :HD]
    z2b_ref[...] = z2[:, HD:]


def _tc_head_body(spa_ref, spb_ref, za_ref, zb_ref, dinv_ref, b_ref, batch_ref,
                  fc1w_ref, fc1b_ref, fc2w_ref, fc2b_ref,
                  out_ref, pool_acc, cnt_acc):
    i = pl.program_id(0)

    @pl.when(i == 0)
    def _():
        pool_acc[...] = jnp.zeros((G, D), jnp.float32)
        cnt_acc[...] = jnp.zeros((G, 8), jnp.float32)

    dinv = dinv_ref[:, 0:1]
    h2 = _combine(spa_ref, spb_ref, za_ref, zb_ref, dinv, b_ref)
    b = batch_ref[0, 0, :]
    oh = (lax.broadcasted_iota(jnp.int32, (G, _R), 0) == b[None, :]
          ).astype(jnp.float32)
    pool_acc[...] += jnp.dot(oh, h2, preferred_element_type=jnp.float32)
    cnt_acc[...] += jnp.broadcast_to(jnp.sum(oh, axis=1, keepdims=True), (G, 8))

    @pl.when(i == _NB - 1)
    def _():
        p = pool_acc[...] / jnp.maximum(cnt_acc[:, 0:1], 1.0)
        p = jnp.dot(p, fc1w_ref[...], preferred_element_type=jnp.float32)
        p = jnp.maximum(p + fc1b_ref[...], 0.0)
        p = jnp.dot(p, fc2w_ref[...], preferred_element_type=jnp.float32)
        out_ref[...] = p + fc2b_ref[...]


def _row_spec(shape):
    return pl.BlockSpec(shape, lambda i: (i, 0))


def _sp_spec():
    return pl.BlockSpec((NC, _R, HD), lambda i: (0, i, 0))


def _full_spec(shape):
    return pl.BlockSpec(shape, lambda i: tuple(0 for _ in shape))


def kernel(x, edge_index, batch, W1, b1, W2, b2, fc1_W, fc1_b, fc2_W, fc2_b):
    src = edge_index[0].reshape(NW, NCHUNK, CHUNK)
    dst = edge_index[1].reshape(NW, NCHUNK, CHUNK)
    ones8 = jnp.ones((CHUNK, 16), jnp.float32)
    zeros8 = jnp.zeros((RPT, 16), jnp.float32)
    zerosh = jnp.zeros((ZR, HD), jnp.float32)

    degp = _sc_degree(dst, ones8, zeros8)

    z1a, z1b, dinv8 = pl.pallas_call(
        _tc_prescale_body,
        grid=(_NB,),
        in_specs=[
            pl.BlockSpec((NC, _R, 16), lambda i: (0, i, 0)),
            _row_spec((_R, D)),
            _full_spec((D, D)),
        ],
        out_specs=[_row_spec((_R, HD)), _row_spec((_R, HD)), _row_spec((_R, 8))],
        out_shape=[
            jax.ShapeDtypeStruct((N, HD), jnp.float32),
            jax.ShapeDtypeStruct((N, HD), jnp.float32),
            jax.ShapeDtypeStruct((N, 8), jnp.float32),
        ],
    )(degp, x, W1)

    sp1a, sp1b = _sc_scatter(z1a, z1b, src, dst, zerosh)

    z2a, z2b = pl.pallas_call(
        _tc_layer_body,
        grid=(_NB,),
        in_specs=[
            _sp_spec(),
            _sp_spec(),
            _row_spec((_R, HD)),
            _row_spec((_R, HD)),
            _row_spec((_R, 8)),
            _full_spec((1, D)),
            _full_spec((D, D)),
        ],
        out_specs=[_row_spec((_R, HD)), _row_spec((_R, HD))],
        out_shape=[
            jax.ShapeDtypeStruct((N, HD), jnp.float32),
            jax.ShapeDtypeStruct((N, HD), jnp.float32),
        ],
    )(sp1a, sp1b, z1a, z1b, dinv8, b1.reshape(1, D), W2)

    sp2a, sp2b = _sc_scatter(z2a, z2b, src, dst, zerosh)

    out = pl.pallas_call(
        _tc_head_body,
        grid=(_NB,),
        in_specs=[
            _sp_spec(),
            _sp_spec(),
            _row_spec((_R, HD)),
            _row_spec((_R, HD)),
            _row_spec((_R, 8)),
            _full_spec((1, D)),
            pl.BlockSpec((1, 1, _R), lambda i: (i, 0, 0)),
            _full_spec((D, D)),
            _full_spec((1, D)),
            _full_spec((D, D)),
            _full_spec((1, D)),
        ],
        out_specs=_full_spec((G, D)),
        out_shape=jax.ShapeDtypeStruct((G, D), jnp.float32),
        scratch_shapes=[
            pltpu.VMEM((G, D), jnp.float32),
            pltpu.VMEM((G, 8), jnp.float32),
        ],
    )(sp2a, sp2b, z2a, z2b, dinv8, b2.reshape(1, D), batch.reshape(_NB, 1, _R),
      fc1_W, fc1_b.reshape(1, D), fc2_W, fc2_b.reshape(1, D))

    return out
